# baseline (speedup 1.0000x reference)
import jax
import jax.numpy as jnp
import numpy as np
from jax.experimental import pallas as pl
from jax.experimental.pallas import tpu as pltpu

_B, _T, _Tc, _D, _V = 2, 256, 1024, 1024, 32000
_VTF = 640
_NV = _V // _VTF


def _fused_body(os_ref, ec_ref, ed_ref, maskf_ref, Wq_ref, Wk_ref, wpg_ref,
                bpg_ref, Wg_ref, bg_ref, ctxT_ref, out_ref,
                m_acc, s_acc, va16, attn16, s_sc):
    p = pl.program_id(0)
    j = pl.program_id(1)
    b = pl.program_id(2)

    @pl.when((p == 0) & (j == 0))
    def _():
        m_acc[b] = jnp.full((_T, 1), -jnp.inf, jnp.float32)
        s_acc[b] = jnp.zeros((_T, 1), jnp.float32)
        os16 = os_ref[b]
        ec16 = ec_ref[b]
        q = jnp.dot(os16, Wq_ref[...], preferred_element_type=jnp.float32)
        k = jnp.dot(ec16, Wk_ref[...], preferred_element_type=jnp.float32)
        scores = jax.lax.dot_general(q.astype(jnp.bfloat16),
                                     k.astype(jnp.bfloat16),
                                     (((1,), (1,)), ((), ())),
                                     preferred_element_type=jnp.float32)
        scores = scores * jnp.float32(1.0 / np.sqrt(_D))
        maskf = maskf_ref[b]
        scores = scores + (1.0 - maskf) * jnp.float32(-1e9)
        m = jnp.max(scores, axis=1, keepdims=True)
        e = jnp.exp(scores - m)
        attn = e / jnp.sum(e, axis=1, keepdims=True)
        attn16[b] = attn.astype(jnp.bfloat16)
        cv = jnp.dot(attn.astype(jnp.bfloat16), ec16,
                     preferred_element_type=jnp.float32)
        wpg = wpg_ref[...]
        slog = (jnp.sum(os16.astype(jnp.float32) * wpg[:, 0:_D],
                        axis=1, keepdims=True)
                + jnp.sum(cv * wpg[:, _D:2 * _D], axis=1, keepdims=True)
                + jnp.sum(ed_ref[b].astype(jnp.float32) * wpg[:, 2 * _D:],
                          axis=1, keepdims=True)
                + bpg_ref[0, 0])
        s_sc[b] = jax.nn.sigmoid(slog)

    @pl.when(p == 0)
    def _():
        va = jax.lax.dot_general(os_ref[b],
                                 Wg_ref[...].astype(jnp.bfloat16),
                                 (((1,), (1,)), ((), ())),
                                 preferred_element_type=jnp.float32)
        va = va + bg_ref[0]
        tm = jnp.max(va, axis=1, keepdims=True)
        new_m = jnp.maximum(m_acc[b], tm)
        s_acc[b] = (s_acc[b] * jnp.exp(m_acc[b] - new_m)
                    + jnp.sum(jnp.exp(va - new_m), axis=1, keepdims=True))
        m_acc[b] = new_m
        va16[b * _NV + j] = va.astype(jnp.bfloat16)

    @pl.when(p == 1)
    def _():
        lse = m_acc[b] + jnp.log(s_acc[b])
        va = va16[b * _NV + j][...].astype(jnp.float32)
        ctx = ctxT_ref[b]
        iota = jax.lax.broadcasted_iota(jnp.int32, (_Tc, _VTF), 1) + j * _VTF
        oh = (ctx == iota).astype(jnp.bfloat16)
        pctx = jnp.dot(attn16[b], oh, preferred_element_type=jnp.float32)
        s = s_sc[b]
        pv = jnp.exp(va - lse)
        out_ref[0] = jnp.log(s * pv + (1.0 - s) * pctx)


def kernel(out_states, encoded_context2, encoded_in_domainslots2, context,
           context_mask, Wg, bg, Wq, Wk, Wpg, bpg):
    nv = _NV
    maskf = context_mask.astype(jnp.float32).reshape(_B, 1, _Tc)
    ctxT = context.astype(jnp.int32).reshape(_B, _Tc, 1)
    bpg2 = bpg.reshape(1, 1)
    bg3 = bg.reshape(nv, 1, _VTF)
    os16 = out_states.astype(jnp.bfloat16)
    ec16 = encoded_context2.astype(jnp.bfloat16)
    ed16 = encoded_in_domainslots2.astype(jnp.bfloat16)
    Wq16 = Wq.astype(jnp.bfloat16)
    Wk16 = Wk.astype(jnp.bfloat16)

    cparams = pltpu.CompilerParams(
        dimension_semantics=("arbitrary", "arbitrary", "arbitrary"))
    out = pl.pallas_call(
        _fused_body,
        grid=(2, nv, _B),
        in_specs=[
            pl.BlockSpec((_B, _T, _D), lambda p, j, b: (0, 0, 0)),
            pl.BlockSpec((_B, _Tc, _D), lambda p, j, b: (0, 0, 0)),
            pl.BlockSpec((_B, _T, _D), lambda p, j, b: (0, 0, 0)),
            pl.BlockSpec((_B, 1, _Tc), lambda p, j, b: (0, 0, 0)),
            pl.BlockSpec((_D, _D), lambda p, j, b: (0, 0)),
            pl.BlockSpec((_D, _D), lambda p, j, b: (0, 0)),
            pl.BlockSpec((1, 3 * _D), lambda p, j, b: (0, 0)),
            pl.BlockSpec((1, 1), lambda p, j, b: (0, 0)),
            pl.BlockSpec((_VTF, _D),
                         lambda p, j, b: (jnp.where(p == 0, j, _NV - 1), 0)),
            pl.BlockSpec((1, 1, _VTF),
                         lambda p, j, b: (jnp.where(p == 0, j, _NV - 1), 0, 0)),
            pl.BlockSpec((_B, _Tc, 1), lambda p, j, b: (0, 0, 0)),
        ],
        out_specs=pl.BlockSpec(
            (1, _T, _VTF),
            lambda p, j, b: (jnp.where(p == 0, 0, b), 0,
                             jnp.where(p == 0, 0, j))),
        out_shape=jax.ShapeDtypeStruct((_B, _T, _V), jnp.float32),
        scratch_shapes=[
            pltpu.VMEM((_B, _T, 1), jnp.float32),
            pltpu.VMEM((_B, _T, 1), jnp.float32),
            pltpu.VMEM((_B * _NV, _T, _VTF), jnp.bfloat16),
            pltpu.VMEM((_B, _T, _Tc), jnp.bfloat16),
            pltpu.VMEM((_B, _T, 1), jnp.float32),
        ],
        compiler_params=cparams,
    )(os16, ec16, ed16, maskf, Wq16, Wk16, Wpg, bpg2, Wg, bg3, ctxT)
    return out
